# baseline (speedup 1.0000x reference)
import functools

import jax
import jax.numpy as jnp
from jax import lax
from jax.experimental import pallas as pl
from jax.experimental.pallas import tpu as pltpu
from jax.experimental.pallas import tpu_sc as plsc

D_MODEL = 768
NUM_EXPERTS = 8
TOP_K = 2
BLOCK = 4096
NSPLIT = 4
SUB = BLOCK // NSPLIT
N_TOK = 32768
NWORKERS = 32
CHUNK = N_TOK // NWORKERS
LANES = 16


def _logits_block(*refs):
    x_refs = refs[:NSPLIT]
    w_ref, b_ref, out_ref = refs[NSPLIT:]
    w = w_ref[...]
    parts = [
        jax.lax.dot_general(
            w, xr[...], (((1,), (1,)), ((), ())),
            preferred_element_type=jnp.float32,
        )
        for xr in x_refs
    ]
    out_ref[...] = jnp.concatenate(parts, axis=1) + b_ref[...]


def _tc_logits(x, W, b):
    n, d = x.shape
    e = W.shape[0]
    b2 = b.reshape(e, 1)
    return pl.pallas_call(
        _logits_block,
        grid=(n // BLOCK,),
        in_specs=[
            pl.BlockSpec((SUB, d), functools.partial(
                lambda j, i: (NSPLIT * i + j, 0), j))
            for j in range(NSPLIT)
        ] + [
            pl.BlockSpec((e, d), lambda i: (0, 0)),
            pl.BlockSpec((e, 1), lambda i: (0, 0)),
        ],
        out_specs=pl.BlockSpec((e, BLOCK), lambda i: (0, i)),
        out_shape=jax.ShapeDtypeStruct((e, n), jnp.float32),
        compiler_params=pltpu.CompilerParams(
            dimension_semantics=("parallel",),
        ),
    )(*([x] * NSPLIT), W, b2)


_SC_MESH = plsc.VectorSubcoreMesh(core_axis_name="c", subcore_axis_name="s")


@functools.partial(
    pl.kernel,
    mesh=_SC_MESH,
    out_type=[
        jax.ShapeDtypeStruct((NUM_EXPERTS, N_TOK), jnp.float32),
        jax.ShapeDtypeStruct((NUM_EXPERTS, N_TOK), jnp.float32),
        jax.ShapeDtypeStruct((TOP_K, N_TOK), jnp.int32),
        jax.ShapeDtypeStruct((TOP_K, N_TOK), jnp.float32),
    ],
    scratch_types=[
        pltpu.VMEM((NUM_EXPERTS, CHUNK), jnp.float32),
        pltpu.VMEM((NUM_EXPERTS, CHUNK), jnp.float32),
        pltpu.VMEM((NUM_EXPERTS, CHUNK), jnp.float32),
        pltpu.VMEM((TOP_K, CHUNK), jnp.int32),
        pltpu.VMEM((TOP_K, CHUNK), jnp.float32),
    ],
)
def _sc_route(logits_hbm, mix_hbm, probs_hbm, idx_hbm, tw_hbm,
              lg_v, mix_v, probs_v, idx_v, tw_v):
    wid = lax.axis_index("s") * 2 + lax.axis_index("c")
    base = wid * CHUNK
    for ex in range(NUM_EXPERTS):
        pltpu.sync_copy(logits_hbm.at[ex, pl.ds(base, CHUNK)], lg_v.at[ex])

    def body(i, carry):
        off = i * LANES
        l = [lg_v[ex, pl.ds(off, LANES)] for ex in range(NUM_EXPERTS)]
        m = l[0]
        for ex in range(1, NUM_EXPERTS):
            m = jnp.maximum(m, l[ex])
        e = [jnp.exp(lv - m) for lv in l]
        s = e[0]
        for ex in range(1, NUM_EXPERTS):
            s = s + e[ex]
        inv_s = 1.0 / s
        v1 = e[0]
        i1 = jnp.zeros((LANES,), jnp.int32)
        for ex in range(1, NUM_EXPERTS):
            upd = e[ex] > v1
            v1 = jnp.where(upd, e[ex], v1)
            i1 = jnp.where(upd, ex, i1)
        neg = jnp.full((LANES,), -1.0, jnp.float32)
        v2 = jnp.where(i1 == 0, neg, e[0])
        i2 = jnp.zeros((LANES,), jnp.int32)
        for ex in range(1, NUM_EXPERTS):
            cand = jnp.where(i1 == ex, neg, e[ex])
            upd = cand > v2
            v2 = jnp.where(upd, cand, v2)
            i2 = jnp.where(upd, ex, i2)
        inv12 = 1.0 / (v1 + v2)
        w1 = v1 * inv12
        w2 = v2 * inv12
        zero = jnp.zeros((LANES,), jnp.float32)
        for ex in range(NUM_EXPERTS):
            probs_v[ex, pl.ds(off, LANES)] = e[ex] * inv_s
            mix_v[ex, pl.ds(off, LANES)] = (
                jnp.where(i1 == ex, w1, zero) + jnp.where(i2 == ex, w2, zero))
        idx_v[0, pl.ds(off, LANES)] = i1
        idx_v[1, pl.ds(off, LANES)] = i2
        tw_v[0, pl.ds(off, LANES)] = w1
        tw_v[1, pl.ds(off, LANES)] = w2
        return carry

    lax.fori_loop(0, CHUNK // LANES, body, 0)

    for ex in range(NUM_EXPERTS):
        pltpu.sync_copy(mix_v.at[ex], mix_hbm.at[ex, pl.ds(base, CHUNK)])
        pltpu.sync_copy(probs_v.at[ex], probs_hbm.at[ex, pl.ds(base, CHUNK)])
    for t in range(TOP_K):
        pltpu.sync_copy(idx_v.at[t], idx_hbm.at[t, pl.ds(base, CHUNK)])
        pltpu.sync_copy(tw_v.at[t], tw_hbm.at[t, pl.ds(base, CHUNK)])


@functools.partial(jax.jit, static_argnames=())
def kernel(x, W, b):
    logits_t = _tc_logits(x, W, b)
    mix_t, probs_t, idx_t, tw_t = _sc_route(logits_t)
    return (mix_t.T, probs_t.T, idx_t.T, tw_t.T)


# speedup vs baseline: 1.7716x; 1.7716x over previous
import functools

import jax
import jax.numpy as jnp
from jax.experimental import pallas as pl
from jax.experimental.pallas import tpu as pltpu

D_MODEL = 768
NUM_EXPERTS = 8
TOP_K = 2
BLOCK = 4096
NSPLIT = 4
SUB = BLOCK // NSPLIT


def _router_block(*refs):
    x_refs = refs[:NSPLIT]
    w_ref, b_ref, mix_ref, probs_ref, idx_ref, tw_ref = refs[NSPLIT:]
    w = w_ref[...]
    parts = [
        jax.lax.dot_general(
            w, xr[...], (((1,), (1,)), ((), ())),
            preferred_element_type=jnp.float32,
        )
        for xr in x_refs
    ]
    logits = jnp.concatenate(parts, axis=1) + b_ref[...]

    m = jnp.max(logits, axis=0, keepdims=True)
    e = jnp.exp(logits - m)
    s = jnp.sum(e, axis=0, keepdims=True)
    probs = e * (1.0 / s)

    erows = jax.lax.broadcasted_iota(jnp.int32, e.shape, 0)
    v1 = jnp.max(e, axis=0, keepdims=True)
    i1 = jnp.min(jnp.where(e == v1, erows, NUM_EXPERTS), axis=0, keepdims=True)
    masked = jnp.where(erows == i1, -1.0, e)
    v2 = jnp.max(masked, axis=0, keepdims=True)
    i2 = jnp.min(jnp.where(masked == v2, erows, NUM_EXPERTS), axis=0, keepdims=True)

    inv = 1.0 / (v1 + v2)
    w1 = v1 * inv
    w2 = v2 * inv

    zero = jnp.zeros_like(e)
    mixing = jnp.where(erows == i1, w1, zero) + jnp.where(erows == i2, w2, zero)

    mix_ref[...] = mixing
    probs_ref[...] = probs
    idx_ref[...] = jnp.concatenate([i1, i2], axis=0)
    tw_ref[...] = jnp.concatenate([w1, w2], axis=0)


@functools.partial(jax.jit, static_argnames=())
def kernel(x, W, b):
    n, d = x.shape
    e = W.shape[0]
    b2 = b.reshape(e, 1)
    grid = (n // BLOCK,)
    out = pl.pallas_call(
        _router_block,
        grid=grid,
        in_specs=[
            pl.BlockSpec((SUB, d), functools.partial(
                lambda j, i: (NSPLIT * i + j, 0), j))
            for j in range(NSPLIT)
        ] + [
            pl.BlockSpec((e, d), lambda i: (0, 0)),
            pl.BlockSpec((e, 1), lambda i: (0, 0)),
        ],
        out_specs=[
            pl.BlockSpec((e, BLOCK), lambda i: (0, i)),
            pl.BlockSpec((e, BLOCK), lambda i: (0, i)),
            pl.BlockSpec((TOP_K, BLOCK), lambda i: (0, i)),
            pl.BlockSpec((TOP_K, BLOCK), lambda i: (0, i)),
        ],
        out_shape=[
            jax.ShapeDtypeStruct((e, n), jnp.float32),
            jax.ShapeDtypeStruct((e, n), jnp.float32),
            jax.ShapeDtypeStruct((TOP_K, n), jnp.int32),
            jax.ShapeDtypeStruct((TOP_K, n), jnp.float32),
        ],
        compiler_params=pltpu.CompilerParams(
            dimension_semantics=("parallel",),
        ),
    )(*([x] * NSPLIT), W, b2)
    mix_t, probs_t, idx_t, tw_t = out
    return (mix_t.T, probs_t.T, idx_t.T, tw_t.T)


# speedup vs baseline: 1.8044x; 1.0185x over previous
import functools

import jax
import jax.numpy as jnp
from jax.experimental import pallas as pl
from jax.experimental.pallas import tpu as pltpu

D_MODEL = 768
NUM_EXPERTS = 8
TOP_K = 2
BLOCK = 4096
NSPLIT = 4
SUB = BLOCK // NSPLIT


def _router_block(*refs):
    x_refs = refs[:NSPLIT]
    w_ref, b_ref, mix_ref, probs_ref, idx_ref, tw_ref = refs[NSPLIT:]
    w = w_ref[...]
    parts = [
        jax.lax.dot_general(
            w, xr[...], (((1,), (1,)), ((), ())),
            preferred_element_type=jnp.float32,
        )
        for xr in x_refs
    ]
    logits = jnp.concatenate(parts, axis=1) + b_ref[...].T

    m = jnp.max(logits, axis=0, keepdims=True)
    e = jnp.exp(logits - m)
    s = jnp.sum(e, axis=0, keepdims=True)
    probs = e * (1.0 / s)

    erows = jax.lax.broadcasted_iota(jnp.int32, e.shape, 0)
    v1 = jnp.max(e, axis=0, keepdims=True)
    i1 = jnp.min(jnp.where(e == v1, erows, NUM_EXPERTS), axis=0, keepdims=True)
    masked = jnp.where(erows == i1, -1.0, e)
    v2 = jnp.max(masked, axis=0, keepdims=True)
    i2 = jnp.min(jnp.where(masked == v2, erows, NUM_EXPERTS), axis=0, keepdims=True)

    inv = 1.0 / (v1 + v2)
    w1 = v1 * inv
    w2 = v2 * inv

    zero = jnp.zeros_like(e)
    mixing = jnp.where(erows == i1, w1, zero) + jnp.where(erows == i2, w2, zero)

    mix_ref[...] = mixing
    probs_ref[...] = probs
    idx_ref[...] = jnp.concatenate([i1, i2], axis=0)
    tw_ref[...] = jnp.concatenate([w1, w2], axis=0)


@functools.partial(jax.jit, static_argnames=())
def kernel(x, W, b):
    n, d = x.shape
    e = W.shape[0]
    b2 = b.reshape(1, e)
    grid = (n // BLOCK,)
    out = pl.pallas_call(
        _router_block,
        grid=grid,
        in_specs=[
            pl.BlockSpec((SUB, d), functools.partial(
                lambda j, i: (NSPLIT * i + j, 0), j))
            for j in range(NSPLIT)
        ] + [
            pl.BlockSpec((e, d), lambda i: (0, 0)),
            pl.BlockSpec((1, e), lambda i: (0, 0)),
        ],
        out_specs=[
            pl.BlockSpec((e, BLOCK), lambda i: (0, i)),
            pl.BlockSpec((e, BLOCK), lambda i: (0, i)),
            pl.BlockSpec((TOP_K, BLOCK), lambda i: (0, i)),
            pl.BlockSpec((TOP_K, BLOCK), lambda i: (0, i)),
        ],
        out_shape=[
            jax.ShapeDtypeStruct((e, n), jnp.float32),
            jax.ShapeDtypeStruct((e, n), jnp.float32),
            jax.ShapeDtypeStruct((TOP_K, n), jnp.int32),
            jax.ShapeDtypeStruct((TOP_K, n), jnp.float32),
        ],
        compiler_params=pltpu.CompilerParams(
            dimension_semantics=("parallel",),
        ),
    )(*([x] * NSPLIT), W, b2)
    mix_t, probs_t, idx_t, tw_t = out
    return (mix_t.T, probs_t.T, idx_t.T, tw_t.T)


# speedup vs baseline: 1.8315x; 1.0150x over previous
import functools

import jax
import jax.numpy as jnp
from jax.experimental import pallas as pl
from jax.experimental.pallas import tpu as pltpu

D_MODEL = 768
NUM_EXPERTS = 8
TOP_K = 2
BLOCK = 4096
NSPLIT = 2
SUB = BLOCK // NSPLIT


def _router_block(*refs):
    x_refs = refs[:NSPLIT]
    w_ref, b_ref, mix_ref, probs_ref, idx_ref, tw_ref = refs[NSPLIT:]
    w = w_ref[...]
    parts = [
        jax.lax.dot_general(
            w, xr[...], (((1,), (1,)), ((), ())),
            preferred_element_type=jnp.float32,
        )
        for xr in x_refs
    ]
    logits = jnp.concatenate(parts, axis=1) + b_ref[...].T

    m = jnp.max(logits, axis=0, keepdims=True)
    e = jnp.exp(logits - m)
    s = jnp.sum(e, axis=0, keepdims=True)
    probs = e * (1.0 / s)

    erows = jax.lax.broadcasted_iota(jnp.int32, e.shape, 0)
    v1 = jnp.max(e, axis=0, keepdims=True)
    i1 = jnp.min(jnp.where(e == v1, erows, NUM_EXPERTS), axis=0, keepdims=True)
    masked = jnp.where(erows == i1, -1.0, e)
    v2 = jnp.max(masked, axis=0, keepdims=True)
    i2 = jnp.min(jnp.where(masked == v2, erows, NUM_EXPERTS), axis=0, keepdims=True)

    inv = 1.0 / (v1 + v2)
    w1 = v1 * inv
    w2 = v2 * inv

    zero = jnp.zeros_like(e)
    mixing = jnp.where(erows == i1, w1, zero) + jnp.where(erows == i2, w2, zero)

    mix_ref[...] = mixing
    probs_ref[...] = probs
    idx_ref[...] = jnp.concatenate([i1, i2], axis=0)
    tw_ref[...] = jnp.concatenate([w1, w2], axis=0)


@functools.partial(jax.jit, static_argnames=())
def kernel(x, W, b):
    n, d = x.shape
    e = W.shape[0]
    b2 = b.reshape(1, e)
    grid = (n // BLOCK,)
    out = pl.pallas_call(
        _router_block,
        grid=grid,
        in_specs=[
            pl.BlockSpec((SUB, d), functools.partial(
                lambda j, i: (NSPLIT * i + j, 0), j))
            for j in range(NSPLIT)
        ] + [
            pl.BlockSpec((e, d), lambda i: (0, 0)),
            pl.BlockSpec((1, e), lambda i: (0, 0)),
        ],
        out_specs=[
            pl.BlockSpec((e, BLOCK), lambda i: (0, i)),
            pl.BlockSpec((e, BLOCK), lambda i: (0, i)),
            pl.BlockSpec((TOP_K, BLOCK), lambda i: (0, i)),
            pl.BlockSpec((TOP_K, BLOCK), lambda i: (0, i)),
        ],
        out_shape=[
            jax.ShapeDtypeStruct((e, n), jnp.float32),
            jax.ShapeDtypeStruct((e, n), jnp.float32),
            jax.ShapeDtypeStruct((TOP_K, n), jnp.int32),
            jax.ShapeDtypeStruct((TOP_K, n), jnp.float32),
        ],
        compiler_params=pltpu.CompilerParams(
            dimension_semantics=("parallel",),
        ),
    )(*([x] * NSPLIT), W, b2)
    mix_t, probs_t, idx_t, tw_t = out
    return (mix_t.T, probs_t.T, idx_t.T, tw_t.T)


# speedup vs baseline: 1.8355x; 1.0022x over previous
import functools

import jax
import jax.numpy as jnp
from jax.experimental import pallas as pl
from jax.experimental.pallas import tpu as pltpu

D_MODEL = 768
NUM_EXPERTS = 8
TOP_K = 2
BLOCK = 4096
NSPLIT = 1
SUB = BLOCK // NSPLIT


def _router_block(*refs):
    x_refs = refs[:NSPLIT]
    w_ref, b_ref, mix_ref, probs_ref, idx_ref, tw_ref = refs[NSPLIT:]
    w = w_ref[...]
    parts = [
        jax.lax.dot_general(
            w, xr[...], (((1,), (1,)), ((), ())),
            preferred_element_type=jnp.float32,
        )
        for xr in x_refs
    ]
    logits = jnp.concatenate(parts, axis=1) + b_ref[...].T

    m = jnp.max(logits, axis=0, keepdims=True)
    e = jnp.exp(logits - m)
    s = jnp.sum(e, axis=0, keepdims=True)
    probs = e * (1.0 / s)

    erows = jax.lax.broadcasted_iota(jnp.int32, e.shape, 0)
    v1 = jnp.max(e, axis=0, keepdims=True)
    i1 = jnp.min(jnp.where(e == v1, erows, NUM_EXPERTS), axis=0, keepdims=True)
    masked = jnp.where(erows == i1, -1.0, e)
    v2 = jnp.max(masked, axis=0, keepdims=True)
    i2 = jnp.min(jnp.where(masked == v2, erows, NUM_EXPERTS), axis=0, keepdims=True)

    inv = 1.0 / (v1 + v2)
    w1 = v1 * inv
    w2 = v2 * inv

    zero = jnp.zeros_like(e)
    mixing = jnp.where(erows == i1, w1, zero) + jnp.where(erows == i2, w2, zero)

    mix_ref[...] = mixing
    probs_ref[...] = probs
    idx_ref[...] = jnp.concatenate([i1, i2], axis=0)
    tw_ref[...] = jnp.concatenate([w1, w2], axis=0)


@functools.partial(jax.jit, static_argnames=())
def kernel(x, W, b):
    n, d = x.shape
    e = W.shape[0]
    b2 = b.reshape(1, e)
    grid = (n // BLOCK,)
    out = pl.pallas_call(
        _router_block,
        grid=grid,
        in_specs=[
            pl.BlockSpec((SUB, d), functools.partial(
                lambda j, i: (NSPLIT * i + j, 0), j))
            for j in range(NSPLIT)
        ] + [
            pl.BlockSpec((e, d), lambda i: (0, 0)),
            pl.BlockSpec((1, e), lambda i: (0, 0)),
        ],
        out_specs=[
            pl.BlockSpec((e, BLOCK), lambda i: (0, i)),
            pl.BlockSpec((e, BLOCK), lambda i: (0, i)),
            pl.BlockSpec((TOP_K, BLOCK), lambda i: (0, i)),
            pl.BlockSpec((TOP_K, BLOCK), lambda i: (0, i)),
        ],
        out_shape=[
            jax.ShapeDtypeStruct((e, n), jnp.float32),
            jax.ShapeDtypeStruct((e, n), jnp.float32),
            jax.ShapeDtypeStruct((TOP_K, n), jnp.int32),
            jax.ShapeDtypeStruct((TOP_K, n), jnp.float32),
        ],
        compiler_params=pltpu.CompilerParams(
            dimension_semantics=("parallel",),
        ),
    )(*([x] * NSPLIT), W, b2)
    mix_t, probs_t, idx_t, tw_t = out
    return (mix_t.T, probs_t.T, idx_t.T, tw_t.T)
